# baseline (speedup 1.0000x reference)
import functools

import jax
import jax.numpy as jnp
from jax import lax
from jax.experimental import pallas as pl
from jax.experimental.pallas import tpu as pltpu
from jax.experimental.pallas import tpu_sc as plsc

N_TASKS = 10000
N_DATA = 10000
N_DEV = 8
E_DT = 160000
E_TT = 160000
E_VT = 80000
H = 4
C = 256
TF = 256
DF = 128
VF = 128
ED_DT = 16
ED_TT = 4
ED_VT = 4

NC = 2
NS = 16
NW = NC * NS
KW = 16
K = NW * KW

_NEG = -1e30


def _ceil_to(x, m):
    return ((x + m - 1) // m) * m


def _ae_pack_body(ea_ref, w_edge_ref, att_e_ref, out_ref):
    f32 = jnp.float32
    ea = ea_ref[...]
    w_edge = w_edge_ref[...]
    att_e = att_e_ref[...]
    a_edge = jnp.concatenate(
        [jnp.sum(w_edge[:, h * C:(h + 1) * C] * att_e[h:h + 1, :], axis=1,
                 keepdims=True) for h in range(H)], axis=1)
    aet = lax.dot_general(a_edge, ea, (((0,), (1,)), ((), ())),
                          preferred_element_type=f32)
    out_ref[...] = aet.reshape(H, 8, 128)


def _ae_pack(ea, w_edge, att_e, e_total, ed):
    f32 = jnp.float32
    ep = _ceil_to(e_total, 1024)
    ea_p = jnp.concatenate(
        [ea, jnp.zeros((ep - e_total, ed), f32)], axis=0)
    out = pl.pallas_call(
        _ae_pack_body,
        grid=(ep // 1024,),
        in_specs=[pl.BlockSpec((1024, ed), lambda i: (i, 0)),
                  pl.BlockSpec((ed, H * C), lambda i: (0, 0)),
                  pl.BlockSpec((H, C), lambda i: (0, 0))],
        out_specs=pl.BlockSpec((H, 8, 128), lambda i: (0, i, 0)),
        out_shape=jax.ShapeDtypeStruct((H, ep // 128, 128), f32),
    )(ea_p, w_edge, att_e)
    return out.reshape(H * (ep // 128), 128)


def _rel_geometry(e_total):
    epw = e_total // NW
    copy_len = epw if epw % 8 == 0 else _ceil_to(epw + 4, 8)
    nchunks = _ceil_to(copy_len, 16) // 16
    ngroups = _ceil_to(nchunks, 16) // 16
    return epw, copy_len, ngroups


def _scan_relation(wid, e_total, dst_v, src_v, ids_comp, eids_comp,
                   idx16, valid_v):
    epw, copy_len, ngroups = _rel_geometry(e_total)
    lo = wid * epw
    lo_a = pl.multiple_of((lo // 8) * 8, 8)
    off = lo - lo_a

    zeros16 = jnp.zeros((16,), jnp.int32)
    ones16 = jnp.full((16,), 1, jnp.int32)
    ids_comp[pl.ds(0, 16)] = zeros16
    ids_comp[pl.ds(16, 16)] = zeros16
    eids_comp[pl.ds(0, 16)] = zeros16
    eids_comp[pl.ds(16, 16)] = zeros16

    lane = lax.iota(jnp.int32, 16)
    kw16 = jnp.full((16,), KW, jnp.int32)
    cap16 = jnp.full((16,), 2 * KW - 1, jnp.int32)

    def splat(sc):
        return jnp.full((16,), sc, jnp.int32)

    def chunk(i, cnt16):
        base = i * 16
        dstv = dst_v[pl.ds(base, 16)]
        srcv = src_v[pl.ds(base, 16)]
        gpos = splat(base) + lane
        m1 = jnp.where(dstv == zeros16, ones16, zeros16)
        m2 = jnp.where(gpos >= splat(off), ones16, zeros16)
        m3 = jnp.where(gpos < splat(off + epw), ones16, zeros16)
        mi = m1 * m2 * m3
        m = mi > zeros16
        incl = jnp.cumsum(mi)
        rincl = lax.rev(jnp.cumsum(lax.rev(mi, (0,))), (0,))
        total16 = incl + rincl - mi
        slot = cnt16 + incl - mi
        slot = jnp.minimum(slot, cap16)
        plsc.store_scatter(ids_comp, (slot,), srcv, mask=m)
        geid = splat(lo_a) + gpos
        plsc.store_scatter(eids_comp, (slot,), geid, mask=m)
        return jnp.minimum(cnt16 + total16, kw16)

    def group(g, cnt16):
        gb = g * 256
        m_or = dst_v[pl.ds(gb, 16)] == zeros16
        for j in range(1, 16):
            m_or = m_or | (dst_v[pl.ds(gb + j * 16, 16)] == zeros16)

        def hit(c):
            return lax.fori_loop(g * 16, g * 16 + 16, chunk, c)

        return lax.cond(jnp.any(m_or), hit, lambda c: c, cnt16)

    cnt16 = lax.fori_loop(0, ngroups, group, zeros16)

    idx16[...] = ids_comp[pl.ds(0, 16)]
    valid_v[...] = jnp.where(lane < cnt16, ones16, zeros16)


def _head_blocks(eids_comp, eidx_h, e_total):
    ep = _ceil_to(e_total, 1024)
    lane = lax.iota(jnp.int32, 16)

    def splat(sc):
        return jnp.full((16,), sc, jnp.int32)

    eid16 = eids_comp[pl.ds(0, 16)]
    blk = lax.shift_right_logical(eid16, splat(7))
    for h in range(H):
        eidx_h[h][...] = blk + splat(h * (ep // 128))


def _extract_ae(eids_comp, earow_h, ea_t_v):
    lane = lax.iota(jnp.int32, 16)

    def splat(sc):
        return jnp.full((16,), sc, jnp.int32)

    eid16 = eids_comp[pl.ds(0, 16)]
    suboff = jnp.bitwise_and(eid16, splat(127))
    for h in range(H):
        ea_t_v[h, :] = plsc.load_gather(earow_h[h], (lane, suboff))


def _sc_body(x_data_hbm, x_tasks_hbm, x_dev_hbm,
             ae_dt_hbm, ae_tt_hbm, ae_vt_hbm,
             dt_src_hbm, dt_dst_hbm, tt_src_hbm, tt_dst_hbm,
             vt_src_hbm, vt_dst_hbm,
             rows_dt_hbm, ea_dt_o_hbm, val_dt_hbm,
             rows_tt_hbm, ea_tt_o_hbm, val_tt_hbm,
             rows_vt_hbm, ea_vt_o_hbm, val_vt_hbm,
             dst_dt, src_dt, dst_tt, src_tt, dst_vt, src_vt,
             ids_dt, eids_dt, idx_dt, vld_dt,
             ids_tt, eids_tt, idx_tt, vld_tt,
             ids_vt, eids_vt, idx_vt, vld_vt,
             rows_dt_v, rows_tt_v, rows_vt_v,
             ex_dt0, ex_dt1, ex_dt2, ex_dt3,
             ex_tt0, ex_tt1, ex_tt2, ex_tt3,
             ex_vt0, ex_vt1, ex_vt2, ex_vt3,
             er_dt0, er_dt1, er_dt2, er_dt3,
             er_tt0, er_tt1, er_tt2, er_tt3,
             er_vt0, er_vt1, er_vt2, er_vt3,
             eat_dt, eat_tt, eat_vt,
             sem_st, sem_g, sem_o):
    wid = lax.axis_index("s") * NC + lax.axis_index("c")

    rels = (
        (E_DT, dt_dst_hbm, dt_src_hbm, x_data_hbm, ae_dt_hbm,
         rows_dt_hbm, ea_dt_o_hbm, val_dt_hbm, dst_dt, src_dt,
         ids_dt, eids_dt, idx_dt, vld_dt, rows_dt_v,
         (ex_dt0, ex_dt1, ex_dt2, ex_dt3), (er_dt0, er_dt1, er_dt2, er_dt3),
         eat_dt),
        (E_TT, tt_dst_hbm, tt_src_hbm, x_tasks_hbm, ae_tt_hbm,
         rows_tt_hbm, ea_tt_o_hbm, val_tt_hbm, dst_tt, src_tt,
         ids_tt, eids_tt, idx_tt, vld_tt, rows_tt_v,
         (ex_tt0, ex_tt1, ex_tt2, ex_tt3), (er_tt0, er_tt1, er_tt2, er_tt3),
         eat_tt),
        (E_VT, vt_dst_hbm, vt_src_hbm, x_dev_hbm, ae_vt_hbm,
         rows_vt_hbm, ea_vt_o_hbm, val_vt_hbm, dst_vt, src_vt,
         ids_vt, eids_vt, idx_vt, vld_vt, rows_vt_v,
         (ex_vt0, ex_vt1, ex_vt2, ex_vt3), (er_vt0, er_vt1, er_vt2, er_vt3),
         eat_vt),
    )

    staged = []
    for (e_total, dst_hbm, src_hbm, _tab, _ae, _orow, _oea, _oval,
         dst_v, src_v, *_rest) in rels:
        epw, copy_len, _ng = _rel_geometry(e_total)
        lo = wid * epw
        lo_a = pl.multiple_of((lo // 8) * 8, 8)
        staged.append(pltpu.async_copy(
            dst_hbm.at[pl.ds(lo_a, copy_len)], dst_v.at[pl.ds(0, copy_len)],
            sem_st))
        staged.append(pltpu.async_copy(
            src_hbm.at[pl.ds(lo_a, copy_len)], src_v.at[pl.ds(0, copy_len)],
            sem_st))
    for d in staged:
        d.wait()

    gathers = []
    for (e_total, _d, _s, table_hbm, ae_hbm, _orow, _oea, _oval,
         dst_v, src_v, ids_comp, eids_comp, idx16, valid_v,
         rows_v, eidx_h, earow_h, _eat) in rels:
        _scan_relation(wid, e_total, dst_v, src_v, ids_comp, eids_comp,
                       idx16, valid_v)
        _head_blocks(eids_comp, eidx_h, e_total)
        gathers.append(pltpu.async_copy(table_hbm.at[idx16], rows_v, sem_g))
        for h in range(H):
            gathers.append(pltpu.async_copy(
                ae_hbm.at[eidx_h[h]], earow_h[h], sem_g))
    for d in gathers:
        d.wait()

    outs = []
    for (e_total, _d, _s, _tab, _ae, out_rows_hbm, out_ea_hbm,
         out_valid_hbm, dst_v, src_v, ids_comp, eids_comp, idx16,
         valid_v, rows_v, eidx_h, earow_h, ea_t_v) in rels:
        _extract_ae(eids_comp, earow_h, ea_t_v)
        outs.append(pltpu.async_copy(
            rows_v, out_rows_hbm.at[pl.ds(wid * KW, KW)], sem_o))
        outs.append(pltpu.async_copy(ea_t_v, out_ea_hbm.at[wid], sem_o))
        outs.append(pltpu.async_copy(
            valid_v, out_valid_hbm.at[pl.ds(wid * KW, KW)], sem_o))
    for d in outs:
        d.wait()


def _sc_compact(x_data, x_tasks, x_devices, ae_dt, ae_tt, ae_vt,
                dt_src, dt_dst, tt_src, tt_dst, vt_src, vt_dst):
    mesh = plsc.VectorSubcoreMesh(core_axis_name="c", subcore_axis_name="s")
    buf = _ceil_to(E_DT // NW, 256)
    f32 = jnp.float32
    i32 = jnp.int32
    out_type = (
        jax.ShapeDtypeStruct((K, DF), f32),
        jax.ShapeDtypeStruct((NW, H, KW), f32),
        jax.ShapeDtypeStruct((K,), i32),
        jax.ShapeDtypeStruct((K, TF), f32),
        jax.ShapeDtypeStruct((NW, H, KW), f32),
        jax.ShapeDtypeStruct((K,), i32),
        jax.ShapeDtypeStruct((K, VF), f32),
        jax.ShapeDtypeStruct((NW, H, KW), f32),
        jax.ShapeDtypeStruct((K,), i32),
    )
    scratch = [
        pltpu.VMEM((buf,), i32), pltpu.VMEM((buf,), i32),
        pltpu.VMEM((buf,), i32), pltpu.VMEM((buf,), i32),
        pltpu.VMEM((buf // 2,), i32), pltpu.VMEM((buf // 2,), i32),
        pltpu.VMEM((32,), i32), pltpu.VMEM((32,), i32),
        pltpu.VMEM((16,), i32), pltpu.VMEM((16,), i32),
        pltpu.VMEM((32,), i32), pltpu.VMEM((32,), i32),
        pltpu.VMEM((16,), i32), pltpu.VMEM((16,), i32),
        pltpu.VMEM((32,), i32), pltpu.VMEM((32,), i32),
        pltpu.VMEM((16,), i32), pltpu.VMEM((16,), i32),
        pltpu.VMEM((KW, DF), f32),
        pltpu.VMEM((KW, TF), f32),
        pltpu.VMEM((KW, VF), f32),
    ] + [pltpu.VMEM((16,), i32) for _ in range(12)] \
      + [pltpu.VMEM((16, 128), f32) for _ in range(12)] \
      + [
        pltpu.VMEM((H, KW), f32),
        pltpu.VMEM((H, KW), f32),
        pltpu.VMEM((H, KW), f32),
        pltpu.SemaphoreType.DMA,
        pltpu.SemaphoreType.DMA,
        pltpu.SemaphoreType.DMA,
    ]
    run = pl.kernel(_sc_body, out_type=out_type, mesh=mesh,
                    scratch_types=scratch,
                    compiler_params=pltpu.CompilerParams(
                        needs_layout_passes=False))
    return run(x_data, x_tasks, x_devices, ae_dt, ae_tt, ae_vt,
               dt_src, dt_dst, tt_src, tt_dst, vt_src, vt_dst)


def _leaky(x, s):
    return jnp.where(x >= 0, x, s * x)


def _relation_out(x0, rows, ea_t, valid_f, w_src, w_dst,
                  att_s, att_d, w_res, bias):
    f32 = jnp.float32
    hs = jnp.dot(rows, w_src, preferred_element_type=f32)
    hd0 = jnp.dot(x0, w_dst, preferred_element_type=f32)
    a_e_all = lax.dot_general(ea_t, jnp.eye(H, dtype=f32),
                              (((0,), (0,)), ((), ())),
                              preferred_element_type=f32)
    acc = jnp.zeros((1, C), f32)
    for h in range(H):
        sl = slice(h * C, (h + 1) * C)
        hs_h = hs[:, sl]
        a_s = jnp.sum(hs_h * att_s[h:h + 1, :], axis=1, keepdims=True)
        a_e = a_e_all[:, h:h + 1]
        a_d = jnp.sum(hd0[:, sl] * att_d[h:h + 1, :], axis=1, keepdims=True)
        alpha = _leaky(a_s + a_e + a_d, 0.2)
        alpha = jnp.where(valid_f > 0.0, alpha, _NEG)
        amax = jnp.max(alpha, axis=0, keepdims=True)
        amax = jnp.where(amax < 0.5 * _NEG, 0.0, amax)
        ex = jnp.exp(alpha - amax) * valid_f
        den = jnp.sum(ex, axis=0, keepdims=True)
        att = ex / (den + 1e-16)
        acc = acc + jnp.sum(hs_h * att, axis=0, keepdims=True)
    out = acc / H + jnp.dot(x0, w_res, preferred_element_type=f32) + bias
    return out


def _layernorm(x, g, b):
    m = jnp.mean(x, axis=1, keepdims=True)
    v = jnp.mean((x - m) * (x - m), axis=1, keepdims=True)
    return (x - m) / jnp.sqrt(v + 1e-5) * g + b


def _tc_body(x0_ref,
             rows_dt, eadt, vdt, rows_tt, eatt, vtt, rows_vt, eavt, vvt,
             ws_dt, wd_dt, as_dt, ad_dt, wr_dt, b_dt,
             ws_tt, wd_tt, as_tt, ad_tt, wr_tt, b_tt,
             ws_vt, wd_vt, as_vt, ad_vt, wr_vt, b_vt,
             ln1_g, ln1_b, fc1_w, fc1_b, ln2_g, ln2_b, fc2_w, fc2_b,
             out_ref):
    x0 = x0_ref[...]
    o1 = _relation_out(x0, rows_dt[...], eadt[...], vdt[...],
                       ws_dt[...], wd_dt[...],
                       as_dt[...], ad_dt[...],
                       wr_dt[...], b_dt[...])
    o2 = _relation_out(x0, rows_tt[...], eatt[...], vtt[...],
                       ws_tt[...], wd_tt[...],
                       as_tt[...], ad_tt[...],
                       wr_tt[...], b_tt[...])
    o3 = _relation_out(x0, rows_vt[...], eavt[...], vvt[...],
                       ws_vt[...], wd_vt[...],
                       as_vt[...], ad_vt[...],
                       wr_vt[...], b_vt[...])
    cand = jnp.concatenate([x0, o1, o2, o3], axis=1)
    x = _leaky(_layernorm(cand, ln1_g[...], ln1_b[...]), 0.01)
    y = jnp.dot(x, fc1_w[...], preferred_element_type=jnp.float32) + fc1_b[...]
    y = _leaky(_layernorm(y, ln2_g[...], ln2_b[...]), 0.01)
    out_ref[...] = (jnp.dot(y, fc2_w[...], preferred_element_type=jnp.float32)
                    + fc2_b[...])


def kernel(x_tasks, x_data, x_devices, ea_dt, ea_tt, ea_vt, params,
           ei_dt_src, ei_dt_dst, ei_tt_src, ei_tt_dst, ei_vt_src, ei_vt_dst):
    i32 = jnp.int32
    f32 = jnp.float32
    pdt0, ptt0, pvt0 = params['dt'], params['tt'], params['vt']
    ae_dt = _ae_pack(ea_dt, pdt0['W_edge'],
                     pdt0['att_edge'].reshape(H, C), E_DT, ED_DT)
    ae_tt = _ae_pack(ea_tt, ptt0['W_edge'],
                     ptt0['att_edge'].reshape(H, C), E_TT, ED_TT)
    ae_vt = _ae_pack(ea_vt, pvt0['W_edge'],
                     pvt0['att_edge'].reshape(H, C), E_VT, ED_VT)
    (rows_dt, eadt, vdt, rows_tt, eatt, vtt, rows_vt, eavt, vvt) = _sc_compact(
        x_data, x_tasks, x_devices, ae_dt, ae_tt, ae_vt,
        ei_dt_src.astype(i32), ei_dt_dst.astype(i32),
        ei_tt_src.astype(i32), ei_tt_dst.astype(i32),
        ei_vt_src.astype(i32), ei_vt_dst.astype(i32))

    x0 = x_tasks[0:1]
    pdt, ptt, pvt = params['dt'], params['tt'], params['vt']

    def prep(p):
        return (p['W_src'], p['W_dst'],
                p['att_src'].reshape(H, C), p['att_dst'].reshape(H, C),
                p['W_res'], p['bias'].reshape(1, C))

    def ea_flat(ea_tiles):
        return jnp.transpose(ea_tiles, (1, 0, 2)).reshape(H, K)

    args = [x0,
            rows_dt, ea_flat(eadt), vdt.reshape(K, 1).astype(f32),
            rows_tt, ea_flat(eatt), vtt.reshape(K, 1).astype(f32),
            rows_vt, ea_flat(eavt), vvt.reshape(K, 1).astype(f32),
            *prep(pdt), *prep(ptt), *prep(pvt),
            params['ln1_g'].reshape(1, -1), params['ln1_b'].reshape(1, -1),
            params['fc1_W'], params['fc1_b'].reshape(1, -1),
            params['ln2_g'].reshape(1, -1), params['ln2_b'].reshape(1, -1),
            params['fc2_W'], params['fc2_b'].reshape(1, -1)]

    out = pl.pallas_call(
        _tc_body,
        out_shape=jax.ShapeDtypeStruct((1, N_DEV), f32),
    )(*args)
    return out.reshape(N_DEV)


# speedup vs baseline: 1.9458x; 1.9458x over previous
import functools

import jax
import jax.numpy as jnp
from jax import lax
from jax.experimental import pallas as pl
from jax.experimental.pallas import tpu as pltpu
from jax.experimental.pallas import tpu_sc as plsc

N_TASKS = 10000
N_DATA = 10000
N_DEV = 8
E_DT = 160000
E_TT = 160000
E_VT = 80000
H = 4
C = 256
TF = 256
DF = 128
VF = 128
ED_DT = 16
ED_TT = 4
ED_VT = 4

NC = 2
NS = 16
NW = NC * NS
KW = 16
K = NW * KW

_NEG = -1e30


def _ceil_to(x, m):
    return ((x + m - 1) // m) * m


def _rel_geometry(e_total):
    epw = e_total // NW
    copy_len = epw if epw % 8 == 0 else _ceil_to(epw + 4, 8)
    nchunks = _ceil_to(copy_len, 16) // 16
    ngroups = _ceil_to(nchunks, 16) // 16
    return epw, copy_len, ngroups


def _scan_relation(wid, e_total, dst_v, src_v, ids_comp, eids_comp,
                   idx16, eidx16, valid_v, ed):
    epw, copy_len, ngroups = _rel_geometry(e_total)
    lo = wid * epw
    lo_a = pl.multiple_of((lo // 8) * 8, 8)
    off = lo - lo_a

    zeros16 = jnp.zeros((16,), jnp.int32)
    ones16 = jnp.full((16,), 1, jnp.int32)
    ids_comp[pl.ds(0, 16)] = zeros16
    ids_comp[pl.ds(16, 16)] = zeros16
    eids_comp[pl.ds(0, 16)] = zeros16
    eids_comp[pl.ds(16, 16)] = zeros16

    lane = lax.iota(jnp.int32, 16)
    kw16 = jnp.full((16,), KW, jnp.int32)
    cap16 = jnp.full((16,), 2 * KW - 1, jnp.int32)

    def splat(sc):
        return jnp.full((16,), sc, jnp.int32)

    def chunk(i, cnt16):
        base = i * 16
        dstv = dst_v[pl.ds(base, 16)]
        srcv = src_v[pl.ds(base, 16)]
        gpos = splat(base) + lane
        m1 = jnp.where(dstv == zeros16, ones16, zeros16)
        m2 = jnp.where(gpos >= splat(off), ones16, zeros16)
        m3 = jnp.where(gpos < splat(off + epw), ones16, zeros16)
        mi = m1 * m2 * m3
        m = mi > zeros16
        incl = jnp.cumsum(mi)
        rincl = lax.rev(jnp.cumsum(lax.rev(mi, (0,))), (0,))
        total16 = incl + rincl - mi
        slot = cnt16 + incl - mi
        slot = jnp.minimum(slot, cap16)
        plsc.store_scatter(ids_comp, (slot,), srcv, mask=m)
        geid = splat(lo_a) + gpos
        plsc.store_scatter(eids_comp, (slot,), geid, mask=m)
        return jnp.minimum(cnt16 + total16, kw16)

    def group(g, cnt16):
        gb = g * 256
        m_or = dst_v[pl.ds(gb, 16)] == zeros16
        for j in range(1, 16):
            m_or = m_or | (dst_v[pl.ds(gb + j * 16, 16)] == zeros16)

        def hit(c):
            return lax.fori_loop(g * 16, g * 16 + 16, chunk, c)

        return lax.cond(jnp.any(m_or), hit, lambda c: c, cnt16)

    cnt16 = lax.fori_loop(0, ngroups, group, zeros16)

    idx16[...] = ids_comp[pl.ds(0, 16)]
    valid_v[...] = jnp.where(lane < cnt16, ones16, zeros16)
    eid16 = eids_comp[pl.ds(0, 16)]
    flat = eid16 * splat(ed)
    eidx16[...] = lax.shift_right_logical(flat, splat(7))


def _extract_ea(eids_comp, earow_v, eaflat_v, ea_t_v, ed):
    lane = lax.iota(jnp.int32, 16)

    def splat(sc):
        return jnp.full((16,), sc, jnp.int32)

    for r in range(16):
        for cc in range(8):
            eaflat_v[pl.ds(r * 128 + cc * 16, 16)] = earow_v[r, pl.ds(cc * 16, 16)]
    eid16 = eids_comp[pl.ds(0, 16)]
    suboff = jnp.bitwise_and(eid16 * splat(ed), splat(127))
    fbase = lane * splat(128) + suboff
    for c in range(ed):
        ea_t_v[c, :] = plsc.load_gather(eaflat_v, (fbase + splat(c),))


def _sc_body(x_data_hbm, x_tasks_hbm, x_dev_hbm,
             ea_dt_hbm, ea_tt_hbm, ea_vt_hbm,
             dt_src_hbm, dt_dst_hbm, tt_src_hbm, tt_dst_hbm,
             vt_src_hbm, vt_dst_hbm,
             rows_dt_hbm, ea_dt_o_hbm, val_dt_hbm,
             rows_tt_hbm, ea_tt_o_hbm, val_tt_hbm,
             rows_vt_hbm, ea_vt_o_hbm, val_vt_hbm,
             dst_dt, src_dt, dst_tt, src_tt, dst_vt, src_vt,
             ids_dt, eids_dt, idx_dt, eidx_dt, vld_dt,
             ids_tt, eids_tt, idx_tt, eidx_tt, vld_tt,
             ids_vt, eids_vt, idx_vt, eidx_vt, vld_vt,
             rows_dt_v, rows_tt_v, rows_vt_v,
             earow_dt, earow_tt, earow_vt, eaflat_v,
             eat_dt, eat_tt, eat_vt,
             sem_st, sem_g, sem_o):
    wid = lax.axis_index("s") * NC + lax.axis_index("c")

    rels = (
        (E_DT, ED_DT, dt_dst_hbm, dt_src_hbm, x_data_hbm, ea_dt_hbm,
         rows_dt_hbm, ea_dt_o_hbm, val_dt_hbm, dst_dt, src_dt,
         ids_dt, eids_dt, idx_dt, eidx_dt, vld_dt, rows_dt_v, earow_dt,
         eat_dt),
        (E_TT, ED_TT, tt_dst_hbm, tt_src_hbm, x_tasks_hbm, ea_tt_hbm,
         rows_tt_hbm, ea_tt_o_hbm, val_tt_hbm, dst_tt, src_tt,
         ids_tt, eids_tt, idx_tt, eidx_tt, vld_tt, rows_tt_v, earow_tt,
         eat_tt),
        (E_VT, ED_VT, vt_dst_hbm, vt_src_hbm, x_dev_hbm, ea_vt_hbm,
         rows_vt_hbm, ea_vt_o_hbm, val_vt_hbm, dst_vt, src_vt,
         ids_vt, eids_vt, idx_vt, eidx_vt, vld_vt, rows_vt_v, earow_vt,
         eat_vt),
    )

    staged = []
    for (e_total, ed, dst_hbm, src_hbm, _tab, _ea, _orow, _oea, _oval,
         dst_v, src_v, *_rest) in rels:
        epw, copy_len, _ng = _rel_geometry(e_total)
        lo = wid * epw
        lo_a = pl.multiple_of((lo // 8) * 8, 8)
        staged.append(pltpu.async_copy(
            dst_hbm.at[pl.ds(lo_a, copy_len)], dst_v.at[pl.ds(0, copy_len)],
            sem_st))
        staged.append(pltpu.async_copy(
            src_hbm.at[pl.ds(lo_a, copy_len)], src_v.at[pl.ds(0, copy_len)],
            sem_st))
    for d in staged:
        d.wait()

    gathers = []
    for (e_total, ed, _d, _s, table_hbm, ea_hbm, _orow, _oea, _oval,
         dst_v, src_v, ids_comp, eids_comp, idx16, eidx16, valid_v,
         rows_v, earow_v, _eat) in rels:
        _scan_relation(wid, e_total, dst_v, src_v, ids_comp, eids_comp,
                       idx16, eidx16, valid_v, ed)
        gathers.append(pltpu.async_copy(table_hbm.at[idx16], rows_v, sem_g))
        gathers.append(pltpu.async_copy(ea_hbm.at[eidx16], earow_v, sem_g))
    for d in gathers:
        d.wait()

    outs = []
    for (e_total, ed, _d, _s, _tab, _ea, out_rows_hbm, out_ea_hbm,
         out_valid_hbm, dst_v, src_v, ids_comp, eids_comp, idx16, eidx16,
         valid_v, rows_v, earow_v, ea_t_v) in rels:
        _extract_ea(eids_comp, earow_v, eaflat_v, ea_t_v, ed)
        outs.append(pltpu.async_copy(
            rows_v, out_rows_hbm.at[pl.ds(wid * KW, KW)], sem_o))
        outs.append(pltpu.async_copy(ea_t_v, out_ea_hbm.at[wid], sem_o))
        outs.append(pltpu.async_copy(
            valid_v, out_valid_hbm.at[pl.ds(wid * KW, KW)], sem_o))
    for d in outs:
        d.wait()


def _sc_compact(x_data, x_tasks, x_devices, ea_dt, ea_tt, ea_vt,
                dt_src, dt_dst, tt_src, tt_dst, vt_src, vt_dst):
    mesh = plsc.VectorSubcoreMesh(core_axis_name="c", subcore_axis_name="s")
    buf = _ceil_to(E_DT // NW, 256)
    f32 = jnp.float32
    i32 = jnp.int32
    out_type = (
        jax.ShapeDtypeStruct((K, DF), f32),
        jax.ShapeDtypeStruct((NW, ED_DT, KW), f32),
        jax.ShapeDtypeStruct((K,), i32),
        jax.ShapeDtypeStruct((K, TF), f32),
        jax.ShapeDtypeStruct((NW, ED_TT, KW), f32),
        jax.ShapeDtypeStruct((K,), i32),
        jax.ShapeDtypeStruct((K, VF), f32),
        jax.ShapeDtypeStruct((NW, ED_VT, KW), f32),
        jax.ShapeDtypeStruct((K,), i32),
    )
    scratch = [
        pltpu.VMEM((buf,), i32), pltpu.VMEM((buf,), i32),
        pltpu.VMEM((buf,), i32), pltpu.VMEM((buf,), i32),
        pltpu.VMEM((buf // 2,), i32), pltpu.VMEM((buf // 2,), i32),
        pltpu.VMEM((32,), i32), pltpu.VMEM((32,), i32),
        pltpu.VMEM((16,), i32), pltpu.VMEM((16,), i32), pltpu.VMEM((16,), i32),
        pltpu.VMEM((32,), i32), pltpu.VMEM((32,), i32),
        pltpu.VMEM((16,), i32), pltpu.VMEM((16,), i32), pltpu.VMEM((16,), i32),
        pltpu.VMEM((32,), i32), pltpu.VMEM((32,), i32),
        pltpu.VMEM((16,), i32), pltpu.VMEM((16,), i32), pltpu.VMEM((16,), i32),
        pltpu.VMEM((KW, DF), f32),
        pltpu.VMEM((KW, TF), f32),
        pltpu.VMEM((KW, VF), f32),
        pltpu.VMEM((16, 128), f32),
        pltpu.VMEM((16, 128), f32),
        pltpu.VMEM((16, 128), f32),
        pltpu.VMEM((2048,), f32),
        pltpu.VMEM((ED_DT, KW), f32),
        pltpu.VMEM((ED_TT, KW), f32),
        pltpu.VMEM((ED_VT, KW), f32),
        pltpu.SemaphoreType.DMA,
        pltpu.SemaphoreType.DMA,
        pltpu.SemaphoreType.DMA,
    ]
    run = pl.kernel(_sc_body, out_type=out_type, mesh=mesh,
                    scratch_types=scratch,
                    compiler_params=pltpu.CompilerParams(
                        needs_layout_passes=False))
    return run(x_data, x_tasks, x_devices,
               ea_dt.reshape(E_DT * ED_DT // 128, 128),
               ea_tt.reshape(E_TT * ED_TT // 128, 128),
               ea_vt.reshape(E_VT * ED_VT // 128, 128),
               dt_src, dt_dst, tt_src, tt_dst, vt_src, vt_dst)


def _leaky(x, s):
    return jnp.where(x >= 0, x, s * x)


def _relation_out(x0, rows, ea_t, valid_f, w_src, w_dst, w_edge,
                  att_s, att_d, att_e, w_res, bias):
    f32 = jnp.float32
    hs = jnp.dot(rows, w_src, preferred_element_type=f32)
    hd0 = jnp.dot(x0, w_dst, preferred_element_type=f32)
    a_edge = jnp.concatenate(
        [jnp.sum(w_edge[:, h * C:(h + 1) * C] * att_e[h:h + 1, :], axis=1,
                 keepdims=True) for h in range(H)], axis=1)
    a_e_all = lax.dot_general(ea_t, a_edge, (((0,), (0,)), ((), ())),
                              preferred_element_type=f32)
    acc = jnp.zeros((1, C), f32)
    for h in range(H):
        sl = slice(h * C, (h + 1) * C)
        hs_h = hs[:, sl]
        a_s = jnp.sum(hs_h * att_s[h:h + 1, :], axis=1, keepdims=True)
        a_e = a_e_all[:, h:h + 1]
        a_d = jnp.sum(hd0[:, sl] * att_d[h:h + 1, :], axis=1, keepdims=True)
        alpha = _leaky(a_s + a_e + a_d, 0.2)
        alpha = jnp.where(valid_f > 0.0, alpha, _NEG)
        amax = jnp.max(alpha, axis=0, keepdims=True)
        amax = jnp.where(amax < 0.5 * _NEG, 0.0, amax)
        ex = jnp.exp(alpha - amax) * valid_f
        den = jnp.sum(ex, axis=0, keepdims=True)
        att = ex / (den + 1e-16)
        acc = acc + jnp.sum(hs_h * att, axis=0, keepdims=True)
    out = acc / H + jnp.dot(x0, w_res, preferred_element_type=f32) + bias
    return out


def _layernorm(x, g, b):
    m = jnp.mean(x, axis=1, keepdims=True)
    v = jnp.mean((x - m) * (x - m), axis=1, keepdims=True)
    return (x - m) / jnp.sqrt(v + 1e-5) * g + b


def _tc_body(x0_ref,
             rows_dt, eadt, vdt, rows_tt, eatt, vtt, rows_vt, eavt, vvt,
             ws_dt, wd_dt, we_dt, as_dt, ad_dt, ae_dt, wr_dt, b_dt,
             ws_tt, wd_tt, we_tt, as_tt, ad_tt, ae_tt, wr_tt, b_tt,
             ws_vt, wd_vt, we_vt, as_vt, ad_vt, ae_vt, wr_vt, b_vt,
             ln1_g, ln1_b, fc1_w, fc1_b, ln2_g, ln2_b, fc2_w, fc2_b,
             out_ref):
    x0 = x0_ref[...]
    o1 = _relation_out(x0, rows_dt[...], eadt[...], vdt[...],
                       ws_dt[...], wd_dt[...], we_dt[...],
                       as_dt[...], ad_dt[...], ae_dt[...],
                       wr_dt[...], b_dt[...])
    o2 = _relation_out(x0, rows_tt[...], eatt[...], vtt[...],
                       ws_tt[...], wd_tt[...], we_tt[...],
                       as_tt[...], ad_tt[...], ae_tt[...],
                       wr_tt[...], b_tt[...])
    o3 = _relation_out(x0, rows_vt[...], eavt[...], vvt[...],
                       ws_vt[...], wd_vt[...], we_vt[...],
                       as_vt[...], ad_vt[...], ae_vt[...],
                       wr_vt[...], b_vt[...])
    cand = jnp.concatenate([x0, o1, o2, o3], axis=1)
    x = _leaky(_layernorm(cand, ln1_g[...], ln1_b[...]), 0.01)
    y = jnp.dot(x, fc1_w[...], preferred_element_type=jnp.float32) + fc1_b[...]
    y = _leaky(_layernorm(y, ln2_g[...], ln2_b[...]), 0.01)
    out_ref[...] = (jnp.dot(y, fc2_w[...], preferred_element_type=jnp.float32)
                    + fc2_b[...])


def kernel(x_tasks, x_data, x_devices, ea_dt, ea_tt, ea_vt, params,
           ei_dt_src, ei_dt_dst, ei_tt_src, ei_tt_dst, ei_vt_src, ei_vt_dst):
    i32 = jnp.int32
    f32 = jnp.float32
    (rows_dt, eadt, vdt, rows_tt, eatt, vtt, rows_vt, eavt, vvt) = _sc_compact(
        x_data, x_tasks, x_devices, ea_dt, ea_tt, ea_vt,
        ei_dt_src.astype(i32), ei_dt_dst.astype(i32),
        ei_tt_src.astype(i32), ei_tt_dst.astype(i32),
        ei_vt_src.astype(i32), ei_vt_dst.astype(i32))

    x0 = x_tasks[0:1]
    pdt, ptt, pvt = params['dt'], params['tt'], params['vt']

    def prep(p):
        return (p['W_src'], p['W_dst'], p['W_edge'],
                p['att_src'].reshape(H, C), p['att_dst'].reshape(H, C),
                p['att_edge'].reshape(H, C), p['W_res'],
                p['bias'].reshape(1, C))

    def ea_flat(ea_tiles, ed):
        return jnp.transpose(ea_tiles, (1, 0, 2)).reshape(ed, K)

    args = [x0,
            rows_dt, ea_flat(eadt, ED_DT), vdt.reshape(K, 1).astype(f32),
            rows_tt, ea_flat(eatt, ED_TT), vtt.reshape(K, 1).astype(f32),
            rows_vt, ea_flat(eavt, ED_VT), vvt.reshape(K, 1).astype(f32),
            *prep(pdt), *prep(ptt), *prep(pvt),
            params['ln1_g'].reshape(1, -1), params['ln1_b'].reshape(1, -1),
            params['fc1_W'], params['fc1_b'].reshape(1, -1),
            params['ln2_g'].reshape(1, -1), params['ln2_b'].reshape(1, -1),
            params['fc2_W'], params['fc2_b'].reshape(1, -1)]

    out = pl.pallas_call(
        _tc_body,
        out_shape=jax.ShapeDtypeStruct((1, N_DEV), f32),
    )(*args)
    return out.reshape(N_DEV)
